# baseline (speedup 1.0000x reference)
import jax
from jax.experimental import pallas as pl
from jax.experimental.pallas import tpu as pltpu

_CHUNK_FRACS = (8, 24, 32)


def _chunk_rows(rows):
    sizes = [rows * f // 64 for f in _CHUNK_FRACS]
    starts = [0]
    for s in sizes[:-1]:
        starts.append(starts[-1] + s)
    return list(zip(starts, sizes))


def _staged_copy(w_ref, o_ref, scratch, in_sems, out_sems):
    chunks = _chunk_rows(o_ref.shape[1])
    in_copies = [
        pltpu.make_async_copy(
            w_ref.at[pl.ds(base, size), :],
            scratch.at[pl.ds(base, size), :],
            in_sems.at[i],
        )
        for i, (base, size) in enumerate(chunks)
    ]
    out_copies = [
        pltpu.make_async_copy(
            scratch.at[pl.ds(base, size), :],
            o_ref.at[0, pl.ds(base, size), :],
            out_sems.at[i],
        )
        for i, (base, size) in enumerate(chunks)
    ]
    for c in in_copies:
        c.start()
    for i in range(len(chunks)):
        in_copies[i].wait()
        out_copies[i].start()
    for c in out_copies:
        c.wait()


def kernel(x, weight):
    seq_len = x.shape[1]
    hidden = weight.shape[1]
    n = len(_CHUNK_FRACS)
    return pl.pallas_call(
        _staged_copy,
        in_specs=[pl.BlockSpec(memory_space=pl.ANY)],
        out_specs=pl.BlockSpec(memory_space=pl.ANY),
        out_shape=jax.ShapeDtypeStruct((1, seq_len, hidden), weight.dtype),
        scratch_shapes=[
            pltpu.VMEM((seq_len, hidden), weight.dtype),
            pltpu.SemaphoreType.DMA((n,)),
            pltpu.SemaphoreType.DMA((n,)),
        ],
    )(weight)


# speedup vs baseline: 1.0471x; 1.0471x over previous
import jax
from jax.experimental import pallas as pl
from jax.experimental.pallas import tpu as pltpu

_CHUNK_FRACS = (8, 48, 8)


def _chunk_rows(rows):
    sizes = [rows * f // 64 for f in _CHUNK_FRACS]
    starts = [0]
    for s in sizes[:-1]:
        starts.append(starts[-1] + s)
    return list(zip(starts, sizes))


def _staged_copy(w_ref, o_ref, scratch, in_sems, out_sems):
    chunks = _chunk_rows(o_ref.shape[1])
    in_copies = [
        pltpu.make_async_copy(
            w_ref.at[pl.ds(base, size), :],
            scratch.at[pl.ds(base, size), :],
            in_sems.at[i],
        )
        for i, (base, size) in enumerate(chunks)
    ]
    out_copies = [
        pltpu.make_async_copy(
            scratch.at[pl.ds(base, size), :],
            o_ref.at[0, pl.ds(base, size), :],
            out_sems.at[i],
        )
        for i, (base, size) in enumerate(chunks)
    ]
    for c in in_copies:
        c.start()
    for i in range(len(chunks)):
        in_copies[i].wait()
        out_copies[i].start()
    for c in out_copies:
        c.wait()


def kernel(x, weight):
    seq_len = x.shape[1]
    hidden = weight.shape[1]
    n = len(_CHUNK_FRACS)
    return pl.pallas_call(
        _staged_copy,
        in_specs=[pl.BlockSpec(memory_space=pl.ANY)],
        out_specs=pl.BlockSpec(memory_space=pl.ANY),
        out_shape=jax.ShapeDtypeStruct((1, seq_len, hidden), weight.dtype),
        scratch_shapes=[
            pltpu.VMEM((seq_len, hidden), weight.dtype),
            pltpu.SemaphoreType.DMA((n,)),
            pltpu.SemaphoreType.DMA((n,)),
        ],
    )(weight)


# speedup vs baseline: 1.0536x; 1.0062x over previous
import jax
from jax.experimental import pallas as pl
from jax.experimental.pallas import tpu as pltpu

_NUM_CHUNKS = 2


def _staged_copy(w_ref, o_ref, scratch, in_sems, out_sems):
    rows = o_ref.shape[1]
    chunk = rows // _NUM_CHUNKS
    in_copies = [
        pltpu.make_async_copy(
            w_ref.at[pl.ds(i * chunk, chunk), :],
            scratch.at[pl.ds(i * chunk, chunk), :],
            in_sems.at[i],
        )
        for i in range(_NUM_CHUNKS)
    ]
    out_copies = [
        pltpu.make_async_copy(
            scratch.at[pl.ds(i * chunk, chunk), :],
            o_ref.at[0, pl.ds(i * chunk, chunk), :],
            out_sems.at[i],
        )
        for i in range(_NUM_CHUNKS)
    ]
    for c in in_copies:
        c.start()
    for i in range(_NUM_CHUNKS):
        in_copies[i].wait()
        out_copies[i].start()
    for c in out_copies:
        c.wait()


def kernel(x, weight):
    seq_len = x.shape[1]
    hidden = weight.shape[1]
    return pl.pallas_call(
        _staged_copy,
        in_specs=[pl.BlockSpec(memory_space=pl.ANY)],
        out_specs=pl.BlockSpec(memory_space=pl.ANY),
        out_shape=jax.ShapeDtypeStruct((1, seq_len, hidden), weight.dtype),
        scratch_shapes=[
            pltpu.VMEM((seq_len, hidden), weight.dtype),
            pltpu.SemaphoreType.DMA((_NUM_CHUNKS,)),
            pltpu.SemaphoreType.DMA((_NUM_CHUNKS,)),
        ],
    )(weight)
